# Initial kernel scaffold; baseline (speedup 1.0000x reference)
import jax
import jax.numpy as jnp
from jax.experimental import pallas as pl


def kernel(reg_logits, dist_targets, pos_mask):
    raise NotImplementedError("write your pallas kernel here")


# speedup vs baseline: 2.7692x; 2.7692x over previous
import functools
import jax
import jax.numpy as jnp
from jax.experimental import pallas as pl
from jax.experimental.pallas import tpu as pltpu

_BINS = 16


def _dfl_body(x_ref, d_ref, m_ref, tot_ref, npos_ref):
    b = pl.program_id(0)
    s = pl.program_id(1)
    x = x_ref[0]
    d = d_ref[0, 0]
    pm = m_ref[0]

    d = jnp.clip(d, 0.0, float(_BINS - 1))
    mx = jnp.max(x, axis=0)
    e = jnp.exp(x - mx[None])
    ssum = jnp.sum(e, axis=0)
    lse = jnp.log(ssum) + mx
    k = jax.lax.broadcasted_iota(jnp.int32, (_BINS, 128, 128), 0).astype(jnp.float32)
    w = jnp.maximum(1.0 - jnp.abs(d[None] - k), 0.0)
    acc = jnp.sum(x * w, axis=0)
    partial = jnp.sum((lse - acc) * pm)

    @pl.when(jnp.logical_and(b == 0, s == 0))
    def _init():
        tot_ref[0, 0] = 0.0
        npos_ref[0, 0] = 0.0

    tot_ref[0, 0] += partial

    @pl.when(s == 0)
    def _count():
        npos_ref[0, 0] += jnp.sum(pm)


@jax.jit
def kernel(reg_logits, dist_targets, pos_mask):
    B, C, H, W = reg_logits.shape
    HW = H * W
    dist_t = jnp.transpose(dist_targets, (0, 2, 1)).reshape(B, 4, H, W)
    pm = pos_mask.astype(jnp.float32).reshape(B, H, W)

    grid = (B, 4)
    tot, npos = pl.pallas_call(
        _dfl_body,
        grid=grid,
        in_specs=[
            pl.BlockSpec((1, _BINS, H, W), lambda b, s: (b, s, 0, 0)),
            pl.BlockSpec((1, 1, H, W), lambda b, s: (b, s, 0, 0)),
            pl.BlockSpec((1, H, W), lambda b, s: (b, 0, 0)),
        ],
        out_specs=[
            pl.BlockSpec(memory_space=pltpu.SMEM),
            pl.BlockSpec(memory_space=pltpu.SMEM),
        ],
        out_shape=[
            jax.ShapeDtypeStruct((1, 1), jnp.float32),
            jax.ShapeDtypeStruct((1, 1), jnp.float32),
        ],
    )(reg_logits, dist_t, pm)

    total = tot[0, 0]
    n_pos = npos[0, 0]
    return jnp.where(n_pos > 0, total / jnp.maximum(n_pos * 4.0, 1.0), 0.0)


# speedup vs baseline: 3.0124x; 1.0878x over previous
import functools
import jax
import jax.numpy as jnp
from jax.experimental import pallas as pl
from jax.experimental.pallas import tpu as pltpu

_BINS = 16


def _dfl_body(x_ref, d_ref, m_ref, tot_ref, npos_ref):
    b = pl.program_id(0)
    s = pl.program_id(1)
    d = jnp.clip(d_ref[0, 0], 0.0, float(_BINS - 1))
    pm = m_ref[0]

    mx = x_ref[0, 0]
    for k in range(1, _BINS):
        mx = jnp.maximum(mx, x_ref[0, k])
    ssum = jnp.zeros((128, 128), jnp.float32)
    for k in range(_BINS):
        ssum += jnp.exp(x_ref[0, k] - mx)
    acc = jnp.zeros((128, 128), jnp.float32)
    for k in range(_BINS):
        acc += x_ref[0, k] * jnp.maximum(1.0 - jnp.abs(d - float(k)), 0.0)
    lse = jnp.log(ssum) + mx
    partial = jnp.sum((lse - acc) * pm)

    @pl.when(jnp.logical_and(b == 0, s == 0))
    def _init():
        tot_ref[0, 0] = 0.0
        npos_ref[0, 0] = 0.0

    tot_ref[0, 0] += partial

    @pl.when(s == 0)
    def _count():
        npos_ref[0, 0] += jnp.sum(pm)


@jax.jit
def kernel(reg_logits, dist_targets, pos_mask):
    B, C, H, W = reg_logits.shape
    HW = H * W
    dist_t = jnp.transpose(dist_targets, (0, 2, 1)).reshape(B, 4, H, W)
    pm = pos_mask.astype(jnp.float32).reshape(B, H, W)

    grid = (B, 4)
    tot, npos = pl.pallas_call(
        _dfl_body,
        grid=grid,
        in_specs=[
            pl.BlockSpec((1, _BINS, H, W), lambda b, s: (b, s, 0, 0)),
            pl.BlockSpec((1, 1, H, W), lambda b, s: (b, s, 0, 0)),
            pl.BlockSpec((1, H, W), lambda b, s: (b, 0, 0)),
        ],
        out_specs=[
            pl.BlockSpec(memory_space=pltpu.SMEM),
            pl.BlockSpec(memory_space=pltpu.SMEM),
        ],
        out_shape=[
            jax.ShapeDtypeStruct((1, 1), jnp.float32),
            jax.ShapeDtypeStruct((1, 1), jnp.float32),
        ],
    )(reg_logits, dist_t, pm)

    total = tot[0, 0]
    n_pos = npos[0, 0]
    return jnp.where(n_pos > 0, total / jnp.maximum(n_pos * 4.0, 1.0), 0.0)


# speedup vs baseline: 4.3669x; 1.4496x over previous
import functools
import jax
import jax.numpy as jnp
from jax.experimental import pallas as pl
from jax.experimental.pallas import tpu as pltpu

_BINS = 16


def _dfl_body(x_ref, d0_ref, d1_ref, d2_ref, d3_ref, m_ref, tot_ref, npos_ref):
    b = pl.program_id(0)
    pm = m_ref[0]
    partial = jnp.zeros((), jnp.float32)
    for s, d_ref in enumerate((d0_ref, d1_ref, d2_ref, d3_ref)):
        d = jnp.clip(d_ref[0], 0.0, float(_BINS - 1))
        base = s * _BINS
        mx = x_ref[0, base]
        for k in range(1, _BINS):
            mx = jnp.maximum(mx, x_ref[0, base + k])
        xp = x_ref[0, base]
        ssum = jnp.exp(xp - mx)
        acc = xp
        for k in range(1, _BINS):
            xk = x_ref[0, base + k]
            ssum += jnp.exp(xk - mx)
            acc += jnp.clip(d - float(k - 1), 0.0, 1.0) * (xk - xp)
            xp = xk
        lse = jnp.log(ssum) + mx
        partial += jnp.sum((lse - acc) * pm)

    @pl.when(b == 0)
    def _init():
        tot_ref[0, 0] = 0.0
        npos_ref[0, 0] = 0.0

    tot_ref[0, 0] += partial
    npos_ref[0, 0] += jnp.sum(pm)


@jax.jit
def kernel(reg_logits, dist_targets, pos_mask):
    B, C, H, W = reg_logits.shape
    ds = [dist_targets[:, :, s].reshape(B, H, W) for s in range(4)]
    pm = pos_mask.astype(jnp.float32).reshape(B, H, W)

    plane = pl.BlockSpec((1, H, W), lambda b: (b, 0, 0))
    tot, npos = pl.pallas_call(
        _dfl_body,
        grid=(B,),
        in_specs=[
            pl.BlockSpec((1, C, H, W), lambda b: (b, 0, 0, 0)),
            plane, plane, plane, plane, plane,
        ],
        out_specs=[
            pl.BlockSpec(memory_space=pltpu.SMEM),
            pl.BlockSpec(memory_space=pltpu.SMEM),
        ],
        out_shape=[
            jax.ShapeDtypeStruct((1, 1), jnp.float32),
            jax.ShapeDtypeStruct((1, 1), jnp.float32),
        ],
    )(reg_logits, *ds, pm)

    total = tot[0, 0]
    n_pos = npos[0, 0]
    return jnp.where(n_pos > 0, total / jnp.maximum(n_pos * 4.0, 1.0), 0.0)


# speedup vs baseline: 4.7603x; 1.0901x over previous
import functools
import jax
import jax.numpy as jnp
from jax.experimental import pallas as pl
from jax.experimental.pallas import tpu as pltpu

_BINS = 16


def _dfl_body(x_ref, d_ref, m_ref, tot_ref, npos_ref):
    b = pl.program_id(0)
    pm = m_ref[0].astype(jnp.float32)
    partial = jnp.zeros((), jnp.float32)
    for s in range(4):
        d = jnp.clip(d_ref[0, s], 0.0, float(_BINS - 1))
        base = s * _BINS
        mx = x_ref[0, base]
        for k in range(1, _BINS):
            mx = jnp.maximum(mx, x_ref[0, base + k])
        xp = x_ref[0, base]
        ssum = jnp.exp(xp - mx)
        acc = xp
        for k in range(1, _BINS):
            xk = x_ref[0, base + k]
            ssum += jnp.exp(xk - mx)
            acc += jnp.clip(d - float(k - 1), 0.0, 1.0) * (xk - xp)
            xp = xk
        lse = jnp.log(ssum) + mx
        partial += jnp.sum((lse - acc) * pm)

    @pl.when(b == 0)
    def _init():
        tot_ref[0, 0] = 0.0
        npos_ref[0, 0] = 0.0

    tot_ref[0, 0] += partial
    npos_ref[0, 0] += jnp.sum(pm)


@jax.jit
def kernel(reg_logits, dist_targets, pos_mask):
    B, C, H, W = reg_logits.shape
    dist_t = jnp.transpose(dist_targets, (0, 2, 1)).reshape(B, 4, H, W)
    pm = pos_mask.reshape(B, H, W)

    tot, npos = pl.pallas_call(
        _dfl_body,
        grid=(B,),
        in_specs=[
            pl.BlockSpec((1, C, H, W), lambda b: (b, 0, 0, 0)),
            pl.BlockSpec((1, 4, H, W), lambda b: (b, 0, 0, 0)),
            pl.BlockSpec((1, H, W), lambda b: (b, 0, 0)),
        ],
        out_specs=[
            pl.BlockSpec(memory_space=pltpu.SMEM),
            pl.BlockSpec(memory_space=pltpu.SMEM),
        ],
        out_shape=[
            jax.ShapeDtypeStruct((1, 1), jnp.float32),
            jax.ShapeDtypeStruct((1, 1), jnp.float32),
        ],
    )(reg_logits, dist_t, pm)

    total = tot[0, 0]
    n_pos = npos[0, 0]
    return jnp.where(n_pos > 0, total / jnp.maximum(n_pos * 4.0, 1.0), 0.0)
